# baseline (speedup 1.0000x reference)
import functools

import jax
import jax.numpy as jnp
from jax import lax
from jax.experimental import pallas as pl
from jax.experimental.pallas import tpu as pltpu

N_SAMPLED = 1024
N_REGION = 32
RADIUS = 0.2
MLP_SIZES = (64, 64, 128)

B = 2
N = 8192
ROW_BLK = 2048


def _fps(n_sample, xyz):
    Bb, Nn, _ = xyz.shape

    def body(i, state):
        min_d2, idx, farthest = state
        idx = idx.at[:, i].set(farthest)
        centroid = jax.vmap(lambda p, f: p[f])(xyz, farthest)
        d2 = jnp.sum((xyz - centroid[:, None, :]) ** 2, axis=-1)
        min_d2 = jnp.minimum(min_d2, d2)
        farthest = jnp.argmax(min_d2, axis=-1).astype(jnp.int32)
        return (min_d2, idx, farthest)

    min_d2 = jnp.full((Bb, Nn), 1e10, dtype=xyz.dtype)
    idx = jnp.zeros((Bb, n_sample), dtype=jnp.int32)
    farthest = jnp.zeros((Bb,), dtype=jnp.int32)
    _, idx, _ = jax.lax.fori_loop(0, n_sample, body, (min_d2, idx, farthest))
    return idx


def _knn(queries, points, k, radius):
    d2 = jnp.sum((queries[:, :, None, :] - points[:, None, :, :]) ** 2, axis=-1)
    neg_d2, idx = jax.lax.top_k(-d2, k)
    dist = jnp.sqrt(jnp.maximum(-neg_d2, 0.0))
    nearest_idx = idx[:, :, :1]
    nearest_dist = dist[:, :, :1]
    mask = dist > radius
    idx = jnp.where(mask, nearest_idx, idx)
    return idx


def _bgather(data, idx):
    return jax.vmap(lambda d, i: d[i])(data, idx)


def _mlp_kernel(h_ref, w1_ref, b1_ref, w2_ref, b2_ref, w3_ref, b3_ref, out_ref):
    h = h_ref[...]
    x = h @ w1_ref[...] + b1_ref[...]
    x = jnp.where(x >= 0, x, 0.2 * x)
    x = x @ w2_ref[...] + b2_ref[...]
    x = jnp.where(x >= 0, x, 0.2 * x)
    x = x @ w3_ref[...] + b3_ref[...]
    x = jnp.where(x >= 0, x, 0.2 * x)
    x = x.reshape(ROW_BLK // N_REGION, N_REGION, 128)
    out_ref[...] = jnp.max(x, axis=1)


def _mlp_maxpool(h_flat, W1, b1, W2, b2, W3, b3):
    rows = h_flat.shape[0]
    grid = rows // ROW_BLK
    out = pl.pallas_call(
        _mlp_kernel,
        grid=(grid,),
        in_specs=[
            pl.BlockSpec((ROW_BLK, h_flat.shape[1]), lambda i: (i, 0)),
            pl.BlockSpec((67, 64), lambda i: (0, 0)),
            pl.BlockSpec((64,), lambda i: (0,)),
            pl.BlockSpec((64, 64), lambda i: (0, 0)),
            pl.BlockSpec((64,), lambda i: (0,)),
            pl.BlockSpec((64, 128), lambda i: (0, 0)),
            pl.BlockSpec((128,), lambda i: (0,)),
        ],
        out_specs=pl.BlockSpec((ROW_BLK // N_REGION, 128), lambda i: (i, 0)),
        out_shape=jax.ShapeDtypeStruct((rows // N_REGION, 128), jnp.float32),
    )(h_flat, W1, b1, W2, b2, W3, b3)
    return out


def kernel(xyz, features, W1, b1, W2, b2, W3, b3):
    sel_idx = _fps(N_SAMPLED, xyz)
    sel_xyz = _bgather(xyz, sel_idx)
    nbr_idx = _knn(sel_xyz, xyz, N_REGION, RADIUS)
    nbr_xyz = _bgather(xyz, nbr_idx) - sel_xyz[:, :, None, :]
    nbr_feat = _bgather(features, nbr_idx)
    h = jnp.concatenate([nbr_xyz, nbr_feat], axis=-1)
    h_flat = h.reshape(B * N_SAMPLED * N_REGION, 67)
    sel_feat = _mlp_maxpool(h_flat, W1, b1, W2, b2, W3, b3)
    sel_feat = sel_feat.reshape(B, N_SAMPLED, 128)
    return (sel_xyz, sel_feat)


# speedup vs baseline: 2.0281x; 2.0281x over previous
import functools

import jax
import jax.numpy as jnp
from jax import lax
from jax.experimental import pallas as pl
from jax.experimental.pallas import tpu as pltpu

N_SAMPLED = 1024
N_REGION = 32
RADIUS = 0.2
MLP_SIZES = (64, 64, 128)

B = 2
N = 8192
ROW_BLK = 2048


_NROW = 64
_SROW = 8


def _fps_kernel(xyzT_ref, selidx_ref):
    iota_n = (
        lax.broadcasted_iota(jnp.int32, (_NROW, 128), 0) * 128
        + lax.broadcasted_iota(jnp.int32, (_NROW, 128), 1)
    )
    iota_s = (
        lax.broadcasted_iota(jnp.int32, (_SROW, 128), 0) * 128
        + lax.broadcasted_iota(jnp.int32, (_SROW, 128), 1)
    )
    x = [[xyzT_ref[b, c] for c in range(3)] for b in range(B)]

    def body(i, state):
        min_d2, far, sel = state
        new_md, new_far, new_sel = [], [], []
        for b in range(B):
            far_b = far[b]
            sel_b = jnp.where(iota_s == i, far_b, sel[b])
            onehot = iota_n == far_b
            d2 = None
            for c in range(3):
                cc = jnp.sum(jnp.where(onehot, x[b][c], 0.0))
                dc = x[b][c] - cc
                sq = dc * dc
                d2 = sq if d2 is None else d2 + sq
            md = jnp.minimum(min_d2[b], d2)
            m = jnp.max(md)
            cand = jnp.where(md == m, iota_n, N)
            new_far.append(jnp.min(cand))
            new_md.append(md)
            new_sel.append(sel_b)
        return (new_md, new_far, new_sel)

    min_d2 = [jnp.full((_NROW, 128), 1e10, dtype=jnp.float32) for _ in range(B)]
    far = [jnp.zeros((), dtype=jnp.int32) for _ in range(B)]
    sel = [jnp.zeros((_SROW, 128), dtype=jnp.int32) for _ in range(B)]
    _, _, sel = lax.fori_loop(0, N_SAMPLED, body, (min_d2, far, sel))
    for b in range(B):
        selidx_ref[b] = sel[b]


def _fps(xyzT):
    sel = pl.pallas_call(
        _fps_kernel,
        out_shape=jax.ShapeDtypeStruct((B, _SROW, 128), jnp.int32),
    )(xyzT)
    return sel.reshape(B, N_SAMPLED)


def _knn(queries, points, k, radius):
    d2 = jnp.sum((queries[:, :, None, :] - points[:, None, :, :]) ** 2, axis=-1)
    neg_d2, idx = jax.lax.top_k(-d2, k)
    dist = jnp.sqrt(jnp.maximum(-neg_d2, 0.0))
    nearest_idx = idx[:, :, :1]
    nearest_dist = dist[:, :, :1]
    mask = dist > radius
    idx = jnp.where(mask, nearest_idx, idx)
    return idx


def _bgather(data, idx):
    return jax.vmap(lambda d, i: d[i])(data, idx)


def _mlp_kernel(h_ref, w1_ref, b1_ref, w2_ref, b2_ref, w3_ref, b3_ref, out_ref):
    h = h_ref[...]
    x = h @ w1_ref[...] + b1_ref[...]
    x = jnp.where(x >= 0, x, 0.2 * x)
    x = x @ w2_ref[...] + b2_ref[...]
    x = jnp.where(x >= 0, x, 0.2 * x)
    x = x @ w3_ref[...] + b3_ref[...]
    x = jnp.where(x >= 0, x, 0.2 * x)
    x = x.reshape(ROW_BLK // N_REGION, N_REGION, 128)
    out_ref[...] = jnp.max(x, axis=1)


def _mlp_maxpool(h_flat, W1, b1, W2, b2, W3, b3):
    rows = h_flat.shape[0]
    grid = rows // ROW_BLK
    out = pl.pallas_call(
        _mlp_kernel,
        grid=(grid,),
        in_specs=[
            pl.BlockSpec((ROW_BLK, h_flat.shape[1]), lambda i: (i, 0)),
            pl.BlockSpec((67, 64), lambda i: (0, 0)),
            pl.BlockSpec((64,), lambda i: (0,)),
            pl.BlockSpec((64, 64), lambda i: (0, 0)),
            pl.BlockSpec((64,), lambda i: (0,)),
            pl.BlockSpec((64, 128), lambda i: (0, 0)),
            pl.BlockSpec((128,), lambda i: (0,)),
        ],
        out_specs=pl.BlockSpec((ROW_BLK // N_REGION, 128), lambda i: (i, 0)),
        out_shape=jax.ShapeDtypeStruct((rows // N_REGION, 128), jnp.float32),
    )(h_flat, W1, b1, W2, b2, W3, b3)
    return out


def kernel(xyz, features, W1, b1, W2, b2, W3, b3):
    xyzT = jnp.transpose(xyz, (0, 2, 1)).reshape(B, 3, _NROW, 128)
    sel_idx = _fps(xyzT)
    sel_xyz = _bgather(xyz, sel_idx)
    nbr_idx = _knn(sel_xyz, xyz, N_REGION, RADIUS)
    nbr_xyz = _bgather(xyz, nbr_idx) - sel_xyz[:, :, None, :]
    nbr_feat = _bgather(features, nbr_idx)
    h = jnp.concatenate([nbr_xyz, nbr_feat], axis=-1)
    h_flat = h.reshape(B * N_SAMPLED * N_REGION, 67)
    sel_feat = _mlp_maxpool(h_flat, W1, b1, W2, b2, W3, b3)
    sel_feat = sel_feat.reshape(B, N_SAMPLED, 128)
    return (sel_xyz, sel_feat)


# speedup vs baseline: 10.6792x; 5.2656x over previous
import functools

import jax
import jax.numpy as jnp
from jax import lax
from jax.experimental import pallas as pl
from jax.experimental.pallas import tpu as pltpu
from jax.experimental.pallas import tpu_sc as plsc

N_SAMPLED = 1024
N_REGION = 32
B = 2
N = 8192
_NROW = 64
_SROW = 8

T_RAD = 0.04000000283122063
BIGF = 3.0e38

NQ = B * N_SAMPLED
NWORK = 32
QPW = NQ // NWORK


def _fps_kernel(xyzT_ref, selidx_ref, selxyz_ref):
    iota_n = (
        lax.broadcasted_iota(jnp.int32, (_NROW, 128), 0) * 128
        + lax.broadcasted_iota(jnp.int32, (_NROW, 128), 1)
    )
    iota_s = (
        lax.broadcasted_iota(jnp.int32, (_SROW, 128), 0) * 128
        + lax.broadcasted_iota(jnp.int32, (_SROW, 128), 1)
    )
    x = [[xyzT_ref[b, c] for c in range(3)] for b in range(B)]

    def body(i, state):
        min_d2, far, sel, sxz = state
        new_md, new_far, new_sel, new_sxz = [], [], [], []
        for b in range(B):
            far_b = far[b]
            sel_b = jnp.where(iota_s == i, far_b, sel[b])
            onehot = iota_n == far_b
            d2 = None
            sxz_b = []
            for c in range(3):
                cc = jnp.sum(jnp.where(onehot, x[b][c], 0.0))
                sxz_b.append(jnp.where(iota_s == i, cc, sxz[b][c]))
                dc = x[b][c] - cc
                sq = dc * dc
                d2 = sq if d2 is None else d2 + sq
            md = jnp.minimum(min_d2[b], d2)
            m = jnp.max(md)
            cand = jnp.where(md == m, iota_n, N)
            new_far.append(jnp.min(cand))
            new_md.append(md)
            new_sel.append(sel_b)
            new_sxz.append(sxz_b)
        return (new_md, new_far, new_sel, new_sxz)

    min_d2 = [jnp.full((_NROW, 128), 1e10, dtype=jnp.float32) for _ in range(B)]
    far = [jnp.zeros((), dtype=jnp.int32) for _ in range(B)]
    sel = [jnp.zeros((_SROW, 128), dtype=jnp.int32) for _ in range(B)]
    sxz = [[jnp.zeros((_SROW, 128), dtype=jnp.float32) for _ in range(3)] for _ in range(B)]
    _, _, sel, sxz = lax.fori_loop(0, N_SAMPLED, body, (min_d2, far, sel, sxz))
    for b in range(B):
        selidx_ref[b] = sel[b]
        for c in range(3):
            selxyz_ref[b, c] = sxz[b][c]


def _fps(xyzT4):
    sel, sxz = pl.pallas_call(
        _fps_kernel,
        out_shape=(
            jax.ShapeDtypeStruct((B, _SROW, 128), jnp.int32),
            jax.ShapeDtypeStruct((B, 3, _SROW, 128), jnp.float32),
        ),
    )(xyzT4)
    return sel.reshape(B, N_SAMPLED), sxz.reshape(B, 3, N_SAMPLED)


def _a_kernel(pf_ref, w1_ref, a_ref):
    a_ref[...] = pf_ref[...] @ w1_ref[...]


def _a_mm(pf, W1p):
    rows = pf.shape[0]
    blk = 2048
    return pl.pallas_call(
        _a_kernel,
        grid=(rows // blk,),
        in_specs=[
            pl.BlockSpec((blk, 67), lambda i: (i, 0)),
            pl.BlockSpec((67, 128), lambda i: (0, 0)),
        ],
        out_specs=pl.BlockSpec((blk, 128), lambda i: (i, 0)),
        out_shape=jax.ShapeDtypeStruct((rows, 128), jnp.float32),
    )(pf, W1p)


_BQ = 256
_BP = 1024


def _d2_kernel(sel_ref, xyzT_ref, out_ref):
    q = sel_ref[0]
    p = xyzT_ref[0]
    d2 = None
    for c in range(3):
        dc = q[:, c : c + 1] - p[c : c + 1, :]
        sq = dc * dc
        d2 = sq if d2 is None else d2 + sq
    out_ref[0] = d2


def _d2mat(sel_xyz, xyzT):
    return pl.pallas_call(
        _d2_kernel,
        grid=(B, N_SAMPLED // _BQ, N // _BP),
        in_specs=[
            pl.BlockSpec((1, _BQ, 3), lambda b, i, j: (b, i, 0)),
            pl.BlockSpec((1, 3, _BP), lambda b, i, j: (b, 0, j)),
        ],
        out_specs=pl.BlockSpec((1, _BQ, _BP), lambda b, i, j: (b, i, j)),
        out_shape=jax.ShapeDtypeStruct((B, N_SAMPLED, N), jnp.float32),
    )(sel_xyz, xyzT)


def _merge16(ad, ai, bd, bi):
    rd = lax.rev(bd, (0,))
    ri = lax.rev(bi, (0,))
    m = ad <= rd
    lod = jnp.where(m, ad, rd)
    loi = jnp.where(m, ai, ri)
    hid = jnp.where(m, rd, ad)
    hii = jnp.where(m, ri, ai)
    lod, loi = plsc.sort_key_val(lod, loi)
    hid, hii = plsc.sort_key_val(hid, hii)
    return lod, loi, hid, hii


def _sc_body(d2_hbm, selidx_hbm, a_hbm, out_hbm, rowbuf, candd, candi, selbuf, idxbuf, hbuf, sem):
    cid = lax.axis_index("c")
    sid = lax.axis_index("s")
    wid = sid * 2 + cid
    pltpu.sync_copy(selidx_hbm, selbuf.at[pl.ds(0, NQ)])
    iota = lax.iota(jnp.int32, 16)
    zeros16 = jnp.zeros((16,), jnp.int32)
    infv = jnp.full((16,), BIGF, jnp.float32)

    def qloop(j, carry):
        q = wid * QPW + j
        pltpu.sync_copy(d2_hbm.at[q], rowbuf)
        sv = selbuf[pl.ds(q, 16)]
        selfv = zeros16 + sv[0]

        def cbody(v, cntv):
            dv = rowbuf[pl.ds(v * 16, 16)]
            mask = dv <= T_RAD
            ones = jnp.where(mask, 1, 0)
            cs = plsc.cumsum(ones)
            pos = cntv + cs - 1
            iv = iota + v * 16
            plsc.store_scatter(candd, [pos], dv, mask=mask)
            plsc.store_scatter(candi, [pos], iv, mask=mask)
            pc = plsc.all_reduce_population_count(mask)
            return cntv + pc

        cntv = lax.fori_loop(0, N // 16, cbody, zeros16)
        cnt = jnp.max(cntv)

        def load_vreg(v):
            cd = candd[pl.ds(v * 16, 16)]
            ci = candi[pl.ds(v * 16, 16)]
            lanei = iota + v * 16
            valid = lanei < cntv
            cd = jnp.where(valid, cd, infv)
            ci = jnp.where(valid, ci, selfv)
            return cd, ci

        def sel_fn():
            c0d, c0i = load_vreg(0)
            c1d, c1i = load_vreg(1)
            s0d, s0i = plsc.sort_key_val(c0d, c0i)
            s1d, s1i = plsc.sort_key_val(c1d, c1i)
            b0d, b0i, b1d, b1i = _merge16(s0d, s0i, s1d, s1i)

            def mbody(v, st):
                b0d, b0i, b1d, b1i = st
                cd, ci = load_vreg(v)
                sd, si = plsc.sort_key_val(cd, ci)
                lo1d, lo1i, hi1d, hi1i = _merge16(b1d, b1i, sd, si)
                b0d, b0i, hi2d, hi2i = _merge16(b0d, b0i, lo1d, lo1i)
                b1d, b1i, _, _ = _merge16(hi2d, hi2i, hi1d, hi1i)
                return (b0d, b0i, b1d, b1i)

            nv = (cnt + 15) // 16
            b0d, b0i, b1d, b1i = lax.fori_loop(2, nv, mbody, (b0d, b0i, b1d, b1i))
            return b0i, b1i

        def nosel_fn():
            _, c0i = load_vreg(0)
            _, c1i = load_vreg(1)
            return c0i, c1i

        b0i, b1i = lax.cond(cnt > N_REGION, sel_fn, nosel_fn)
        boff = (q // N_SAMPLED) * N
        idxbuf[0:16] = b0i + boff
        idxbuf[16:32] = b1i + boff
        pltpu.async_copy(a_hbm.at[idxbuf], hbuf, sem).wait()
        pltpu.sync_copy(hbuf, out_hbm.at[q])
        return carry

    lax.fori_loop(0, QPW, qloop, 0)


def _sel_gather(d2_flat, selidx_flat, a_flat):
    mesh = plsc.VectorSubcoreMesh(core_axis_name="c", subcore_axis_name="s")
    f = functools.partial(
        pl.kernel,
        mesh=mesh,
        compiler_params=pltpu.CompilerParams(needs_layout_passes=False),
        out_type=jax.ShapeDtypeStruct((NQ, N_REGION, 128), jnp.float32),
        scratch_types=[
            pltpu.VMEM((N,), jnp.float32),
            pltpu.VMEM((N + 32,), jnp.float32),
            pltpu.VMEM((N + 32,), jnp.int32),
            pltpu.VMEM((NQ + 16,), jnp.int32),
            pltpu.VMEM((N_REGION,), jnp.int32),
            pltpu.VMEM((N_REGION, 128), jnp.float32),
            pltpu.SemaphoreType.DMA,
        ],
    )(_sc_body)
    return f(d2_flat, selidx_flat, a_flat)


_QBLK = 64


def _mlp_kernel(h_ref, sxz_ref, w1x_ref, b1_ref, w2_ref, b2_ref, w3_ref, b3_ref, out_ref):
    dq = b1_ref[...] - sxz_ref[...] @ w1x_ref[...]
    x = h_ref[...][:, :, 0:64] + dq[:, None, :]
    x = jnp.where(x >= 0, x, 0.2 * x)
    x = x.reshape(_QBLK * N_REGION, 64)
    x = x @ w2_ref[...] + b2_ref[...]
    x = jnp.where(x >= 0, x, 0.2 * x)
    x = x @ w3_ref[...] + b3_ref[...]
    x = jnp.where(x >= 0, x, 0.2 * x)
    x = x.reshape(_QBLK, N_REGION, 128)
    out_ref[...] = jnp.max(x, axis=1)


def _mlp(H, sel_flat, W1x, b1, W2, b2, W3, b3):
    return pl.pallas_call(
        _mlp_kernel,
        grid=(NQ // _QBLK,),
        in_specs=[
            pl.BlockSpec((_QBLK, N_REGION, 128), lambda i: (i, 0, 0)),
            pl.BlockSpec((_QBLK, 3), lambda i: (i, 0)),
            pl.BlockSpec((3, 64), lambda i: (0, 0)),
            pl.BlockSpec((64,), lambda i: (0,)),
            pl.BlockSpec((64, 64), lambda i: (0, 0)),
            pl.BlockSpec((64,), lambda i: (0,)),
            pl.BlockSpec((64, 128), lambda i: (0, 0)),
            pl.BlockSpec((128,), lambda i: (0,)),
        ],
        out_specs=pl.BlockSpec((_QBLK, 128), lambda i: (i, 0)),
        out_shape=jax.ShapeDtypeStruct((NQ, 128), jnp.float32),
    )(H, sel_flat, W1x, b1, W2, b2, W3, b3)


def kernel(xyz, features, W1, b1, W2, b2, W3, b3):
    xyzT = jnp.transpose(xyz, (0, 2, 1))
    sel_idx, sel_xyzT = _fps(xyzT.reshape(B, 3, _NROW, 128))
    sel_xyz = jnp.transpose(sel_xyzT, (0, 2, 1))
    pf = jnp.concatenate([xyz, features], axis=-1).reshape(B * N, 67)
    W1p = jnp.pad(W1, ((0, 0), (0, 64)))
    A = _a_mm(pf, W1p)
    d2 = _d2mat(sel_xyz, xyzT)
    H = _sel_gather(d2.reshape(NQ, N), sel_idx.reshape(NQ), A)
    sel_feat = _mlp(H, sel_xyz.reshape(NQ, 3), W1[:3], b1, W2, b2, W3, b3)
    return (sel_xyz, sel_feat.reshape(B, N_SAMPLED, 128))


# speedup vs baseline: 11.3139x; 1.0594x over previous
import functools

import jax
import jax.numpy as jnp
from jax import lax
from jax.experimental import pallas as pl
from jax.experimental.pallas import tpu as pltpu
from jax.experimental.pallas import tpu_sc as plsc

N_SAMPLED = 1024
N_REGION = 32
B = 2
N = 8192
_NROW = 64
_SROW = 8

T_RAD = 0.04000000283122063
BIGF = 3.0e38

NQ = B * N_SAMPLED
NWORK = 32
QPW = NQ // NWORK


def _fps_kernel(xyzT_ref, selidx_ref, selxyz_ref):
    iota_n = (
        lax.broadcasted_iota(jnp.int32, (_NROW, 128), 0) * 128
        + lax.broadcasted_iota(jnp.int32, (_NROW, 128), 1)
    )
    iota_s = (
        lax.broadcasted_iota(jnp.int32, (_SROW, 128), 0) * 128
        + lax.broadcasted_iota(jnp.int32, (_SROW, 128), 1)
    )
    x = [[xyzT_ref[b, c] for c in range(3)] for b in range(B)]

    def body(i, state):
        min_d2, far, sel, sxz = state
        new_md, new_far, new_sel, new_sxz = [], [], [], []
        for b in range(B):
            far_b = far[b]
            sel_b = jnp.where(iota_s == i, far_b, sel[b])
            onehot = iota_n == far_b
            d2 = None
            sxz_b = []
            for c in range(3):
                cc = jnp.sum(jnp.where(onehot, x[b][c], 0.0))
                sxz_b.append(jnp.where(iota_s == i, cc, sxz[b][c]))
                dc = x[b][c] - cc
                sq = dc * dc
                d2 = sq if d2 is None else d2 + sq
            md = jnp.minimum(min_d2[b], d2)
            m = jnp.max(md)
            cand = jnp.where(md == m, iota_n, N)
            new_far.append(jnp.min(cand))
            new_md.append(md)
            new_sel.append(sel_b)
            new_sxz.append(sxz_b)
        return (new_md, new_far, new_sel, new_sxz)

    min_d2 = [jnp.full((_NROW, 128), 1e10, dtype=jnp.float32) for _ in range(B)]
    far = [jnp.zeros((), dtype=jnp.int32) for _ in range(B)]
    sel = [jnp.zeros((_SROW, 128), dtype=jnp.int32) for _ in range(B)]
    sxz = [[jnp.zeros((_SROW, 128), dtype=jnp.float32) for _ in range(3)] for _ in range(B)]
    _, _, sel, sxz = lax.fori_loop(0, N_SAMPLED, body, (min_d2, far, sel, sxz))
    for b in range(B):
        selidx_ref[b] = sel[b]
        for c in range(3):
            selxyz_ref[b, c] = sxz[b][c]


def _fps(xyzT4):
    sel, sxz = pl.pallas_call(
        _fps_kernel,
        out_shape=(
            jax.ShapeDtypeStruct((B, _SROW, 128), jnp.int32),
            jax.ShapeDtypeStruct((B, 3, _SROW, 128), jnp.float32),
        ),
    )(xyzT4)
    return sel.reshape(B, N_SAMPLED), sxz.reshape(B, 3, N_SAMPLED)


def _a_kernel(pf_ref, w1_ref, a_ref):
    a_ref[...] = pf_ref[...] @ w1_ref[...]


def _a_mm(pf, W1p):
    rows = pf.shape[0]
    blk = 2048
    return pl.pallas_call(
        _a_kernel,
        grid=(rows // blk,),
        in_specs=[
            pl.BlockSpec((blk, 67), lambda i: (i, 0)),
            pl.BlockSpec((67, 128), lambda i: (0, 0)),
        ],
        out_specs=pl.BlockSpec((blk, 128), lambda i: (i, 0)),
        out_shape=jax.ShapeDtypeStruct((rows, 128), jnp.float32),
    )(pf, W1p)


_BQ = 256
_BP = 1024


def _d2_kernel(sel_ref, xyzT_ref, out_ref):
    q = sel_ref[0]
    p = xyzT_ref[0]
    d2 = None
    for c in range(3):
        dc = q[:, c : c + 1] - p[c : c + 1, :]
        sq = dc * dc
        d2 = sq if d2 is None else d2 + sq
    out_ref[0] = d2


def _d2mat(sel_xyz, xyzT):
    return pl.pallas_call(
        _d2_kernel,
        grid=(B, N_SAMPLED // _BQ, N // _BP),
        in_specs=[
            pl.BlockSpec((1, _BQ, 3), lambda b, i, j: (b, i, 0)),
            pl.BlockSpec((1, 3, _BP), lambda b, i, j: (b, 0, j)),
        ],
        out_specs=pl.BlockSpec((1, _BQ, _BP), lambda b, i, j: (b, i, j)),
        out_shape=jax.ShapeDtypeStruct((B, N_SAMPLED, N), jnp.float32),
    )(sel_xyz, xyzT)


def _merge16(ad, ai, bd, bi):
    rd = lax.rev(bd, (0,))
    ri = lax.rev(bi, (0,))
    m = ad <= rd
    lod = jnp.where(m, ad, rd)
    loi = jnp.where(m, ai, ri)
    hid = jnp.where(m, rd, ad)
    hii = jnp.where(m, ri, ai)
    lod, loi = plsc.sort_key_val(lod, loi)
    hid, hii = plsc.sort_key_val(hid, hii)
    return lod, loi, hid, hii


def _sc_body(d2_hbm, selidx_hbm, a_hbm, out_hbm, rowbuf, candd, candi, selbuf, idxbuf, hbuf, sem):
    cid = lax.axis_index("c")
    sid = lax.axis_index("s")
    wid = sid * 2 + cid
    pltpu.sync_copy(selidx_hbm, selbuf.at[pl.ds(0, NQ)])
    iota = lax.iota(jnp.int32, 16)
    zeros16 = jnp.zeros((16,), jnp.int32)
    infv = jnp.full((16,), BIGF, jnp.float32)

    def qloop(j, carry):
        q = wid * QPW + j
        pltpu.sync_copy(d2_hbm.at[q], rowbuf)
        sv = selbuf[pl.ds(q, 16)]
        selfv = zeros16 + sv[0]

        def cbody(v, cnt):
            for u in range(2):
                off = (2 * v + u) * 16
                dv = rowbuf[pl.ds(off, 16)]
                mask = dv <= T_RAD
                plsc.store_compressed(candd.at[pl.ds(cnt, 16)], dv, mask=mask)
                iv = iota + off
                plsc.store_compressed(candi.at[pl.ds(cnt, 16)], iv, mask=mask)
                pc = plsc.all_reduce_population_count(mask)
                cnt = cnt + pc[0]
            return cnt

        cnt = lax.fori_loop(0, N // 32, cbody, jnp.zeros((), jnp.int32))
        cntv = zeros16 + cnt

        def load_vreg(v):
            cd = candd[pl.ds(v * 16, 16)]
            ci = candi[pl.ds(v * 16, 16)]
            lanei = iota + v * 16
            valid = lanei < cntv
            cd = jnp.where(valid, cd, infv)
            ci = jnp.where(valid, ci, selfv)
            return cd, ci

        def sel_fn():
            c0d, c0i = load_vreg(0)
            c1d, c1i = load_vreg(1)
            s0d, s0i = plsc.sort_key_val(c0d, c0i)
            s1d, s1i = plsc.sort_key_val(c1d, c1i)
            b0d, b0i, b1d, b1i = _merge16(s0d, s0i, s1d, s1i)

            def mbody(v, st):
                b0d, b0i, b1d, b1i = st
                cd, ci = load_vreg(v)
                sd, si = plsc.sort_key_val(cd, ci)
                lo1d, lo1i, hi1d, hi1i = _merge16(b1d, b1i, sd, si)
                b0d, b0i, hi2d, hi2i = _merge16(b0d, b0i, lo1d, lo1i)
                b1d, b1i, _, _ = _merge16(hi2d, hi2i, hi1d, hi1i)
                return (b0d, b0i, b1d, b1i)

            nv = (cnt + 15) // 16
            b0d, b0i, b1d, b1i = lax.fori_loop(2, nv, mbody, (b0d, b0i, b1d, b1i))
            return b0i, b1i

        def nosel_fn():
            _, c0i = load_vreg(0)
            _, c1i = load_vreg(1)
            return c0i, c1i

        b0i, b1i = lax.cond(cnt > N_REGION, sel_fn, nosel_fn)
        boff = (q // N_SAMPLED) * N
        idxbuf[0:16] = b0i + boff
        idxbuf[16:32] = b1i + boff
        pltpu.async_copy(a_hbm.at[idxbuf], hbuf, sem).wait()
        pltpu.sync_copy(hbuf, out_hbm.at[q])
        return carry

    lax.fori_loop(0, QPW, qloop, 0)


def _sel_gather(d2_flat, selidx_flat, a_flat):
    mesh = plsc.VectorSubcoreMesh(core_axis_name="c", subcore_axis_name="s")
    f = functools.partial(
        pl.kernel,
        mesh=mesh,
        compiler_params=pltpu.CompilerParams(needs_layout_passes=False),
        out_type=jax.ShapeDtypeStruct((NQ, N_REGION, 128), jnp.float32),
        scratch_types=[
            pltpu.VMEM((N,), jnp.float32),
            pltpu.VMEM((N + 32,), jnp.float32),
            pltpu.VMEM((N + 32,), jnp.int32),
            pltpu.VMEM((NQ + 16,), jnp.int32),
            pltpu.VMEM((N_REGION,), jnp.int32),
            pltpu.VMEM((N_REGION, 128), jnp.float32),
            pltpu.SemaphoreType.DMA,
        ],
    )(_sc_body)
    return f(d2_flat, selidx_flat, a_flat)


_QBLK = 64


def _mlp_kernel(h_ref, sxz_ref, w1x_ref, b1_ref, w2_ref, b2_ref, w3_ref, b3_ref, out_ref):
    dq = b1_ref[...] - sxz_ref[...] @ w1x_ref[...]
    x = h_ref[...][:, :, 0:64] + dq[:, None, :]
    x = jnp.where(x >= 0, x, 0.2 * x)
    x = x.reshape(_QBLK * N_REGION, 64)
    x = x @ w2_ref[...] + b2_ref[...]
    x = jnp.where(x >= 0, x, 0.2 * x)
    x = x @ w3_ref[...] + b3_ref[...]
    x = jnp.where(x >= 0, x, 0.2 * x)
    x = x.reshape(_QBLK, N_REGION, 128)
    out_ref[...] = jnp.max(x, axis=1)


def _mlp(H, sel_flat, W1x, b1, W2, b2, W3, b3):
    return pl.pallas_call(
        _mlp_kernel,
        grid=(NQ // _QBLK,),
        in_specs=[
            pl.BlockSpec((_QBLK, N_REGION, 128), lambda i: (i, 0, 0)),
            pl.BlockSpec((_QBLK, 3), lambda i: (i, 0)),
            pl.BlockSpec((3, 64), lambda i: (0, 0)),
            pl.BlockSpec((64,), lambda i: (0,)),
            pl.BlockSpec((64, 64), lambda i: (0, 0)),
            pl.BlockSpec((64,), lambda i: (0,)),
            pl.BlockSpec((64, 128), lambda i: (0, 0)),
            pl.BlockSpec((128,), lambda i: (0,)),
        ],
        out_specs=pl.BlockSpec((_QBLK, 128), lambda i: (i, 0)),
        out_shape=jax.ShapeDtypeStruct((NQ, 128), jnp.float32),
    )(H, sel_flat, W1x, b1, W2, b2, W3, b3)


def kernel(xyz, features, W1, b1, W2, b2, W3, b3):
    xyzT = jnp.transpose(xyz, (0, 2, 1))
    sel_idx, sel_xyzT = _fps(xyzT.reshape(B, 3, _NROW, 128))
    sel_xyz = jnp.transpose(sel_xyzT, (0, 2, 1))
    pf = jnp.concatenate([xyz, features], axis=-1).reshape(B * N, 67)
    W1p = jnp.pad(W1, ((0, 0), (0, 64)))
    A = _a_mm(pf, W1p)
    d2 = _d2mat(sel_xyz, xyzT)
    H = _sel_gather(d2.reshape(NQ, N), sel_idx.reshape(NQ), A)
    sel_feat = _mlp(H, sel_xyz.reshape(NQ, 3), W1[:3], b1, W2, b2, W3, b3)
    return (sel_xyz, sel_feat.reshape(B, N_SAMPLED, 128))


# speedup vs baseline: 11.4244x; 1.0098x over previous
import functools

import jax
import jax.numpy as jnp
from jax import lax
from jax.experimental import pallas as pl
from jax.experimental.pallas import tpu as pltpu
from jax.experimental.pallas import tpu_sc as plsc

N_SAMPLED = 1024
N_REGION = 32
B = 2
N = 8192
_NROW = 64
_SROW = 8

T_RAD = 0.04000000283122063
BIGF = 3.0e38

NQ = B * N_SAMPLED
NWORK = 32
QPW = NQ // NWORK


def _fps_kernel(xyzT_ref, xyzr_ref, selidx_ref, selxyz_ref):
    iota_n = (
        lax.broadcasted_iota(jnp.int32, (_NROW, 128), 0) * 128
        + lax.broadcasted_iota(jnp.int32, (_NROW, 128), 1)
    )
    iota_s = (
        lax.broadcasted_iota(jnp.int32, (_SROW, 128), 0) * 128
        + lax.broadcasted_iota(jnp.int32, (_SROW, 128), 1)
    )
    x = [[xyzT_ref[b, c] for c in range(3)] for b in range(B)]

    def body(i, state):
        min_d2, far, sel, sxz = state
        new_md, new_far, new_sel, new_sxz = [], [], [], []
        for b in range(B):
            far_b = far[b]
            sel_b = jnp.where(iota_s == i, far_b, sel[b])
            cvec = xyzr_ref[b, pl.ds(far_b, 1), :]
            d2 = None
            sxz_b = []
            for c in range(3):
                cc = cvec[0:1, c : c + 1]
                sxz_b.append(jnp.where(iota_s == i, cc, sxz[b][c]))
                dc = x[b][c] - cc
                sq = dc * dc
                d2 = sq if d2 is None else d2 + sq
            md = jnp.minimum(min_d2[b], d2)
            m = jnp.max(md)
            cand = jnp.where(md == m, iota_n, N)
            new_far.append(jnp.min(cand))
            new_md.append(md)
            new_sel.append(sel_b)
            new_sxz.append(sxz_b)
        return (new_md, new_far, new_sel, new_sxz)

    min_d2 = [jnp.full((_NROW, 128), 1e10, dtype=jnp.float32) for _ in range(B)]
    far = [jnp.zeros((), dtype=jnp.int32) for _ in range(B)]
    sel = [jnp.zeros((_SROW, 128), dtype=jnp.int32) for _ in range(B)]
    sxz = [[jnp.zeros((_SROW, 128), dtype=jnp.float32) for _ in range(3)] for _ in range(B)]
    _, _, sel, sxz = lax.fori_loop(0, N_SAMPLED, body, (min_d2, far, sel, sxz))
    for b in range(B):
        selidx_ref[b] = sel[b]
        for c in range(3):
            selxyz_ref[b, c] = sxz[b][c]


def _fps(xyzT4, xyz):
    sel, sxz = pl.pallas_call(
        _fps_kernel,
        out_shape=(
            jax.ShapeDtypeStruct((B, _SROW, 128), jnp.int32),
            jax.ShapeDtypeStruct((B, 3, _SROW, 128), jnp.float32),
        ),
    )(xyzT4, xyz)
    return sel.reshape(B, N_SAMPLED), sxz.reshape(B, 3, N_SAMPLED)


def _a_kernel(pf_ref, w1_ref, a_ref):
    a_ref[...] = pf_ref[...] @ w1_ref[...]


def _a_mm(pf, W1p):
    rows = pf.shape[0]
    blk = 2048
    return pl.pallas_call(
        _a_kernel,
        grid=(rows // blk,),
        in_specs=[
            pl.BlockSpec((blk, 67), lambda i: (i, 0)),
            pl.BlockSpec((67, 128), lambda i: (0, 0)),
        ],
        out_specs=pl.BlockSpec((blk, 128), lambda i: (i, 0)),
        out_shape=jax.ShapeDtypeStruct((rows, 128), jnp.float32),
    )(pf, W1p)


_BQ = 256
_BP = 1024


def _d2_kernel(sel_ref, xyzT_ref, out_ref):
    q = sel_ref[0]
    p = xyzT_ref[0]
    d2 = None
    for c in range(3):
        dc = q[:, c : c + 1] - p[c : c + 1, :]
        sq = dc * dc
        d2 = sq if d2 is None else d2 + sq
    out_ref[0] = d2


def _d2mat(sel_xyz, xyzT):
    return pl.pallas_call(
        _d2_kernel,
        grid=(B, N_SAMPLED // _BQ, N // _BP),
        in_specs=[
            pl.BlockSpec((1, _BQ, 3), lambda b, i, j: (b, i, 0)),
            pl.BlockSpec((1, 3, _BP), lambda b, i, j: (b, 0, j)),
        ],
        out_specs=pl.BlockSpec((1, _BQ, _BP), lambda b, i, j: (b, i, j)),
        out_shape=jax.ShapeDtypeStruct((B, N_SAMPLED, N), jnp.float32),
    )(sel_xyz, xyzT)


def _merge16(ad, ai, bd, bi):
    rd = lax.rev(bd, (0,))
    ri = lax.rev(bi, (0,))
    m = ad <= rd
    lod = jnp.where(m, ad, rd)
    loi = jnp.where(m, ai, ri)
    hid = jnp.where(m, rd, ad)
    hii = jnp.where(m, ri, ai)
    lod, loi = plsc.sort_key_val(lod, loi)
    hid, hii = plsc.sort_key_val(hid, hii)
    return lod, loi, hid, hii


def _sc_body(d2_hbm, selidx_hbm, a_hbm, out_hbm, rowbuf, candd, candi, selbuf, idxbuf, hbuf, sem):
    cid = lax.axis_index("c")
    sid = lax.axis_index("s")
    wid = sid * 2 + cid
    pltpu.sync_copy(selidx_hbm, selbuf.at[pl.ds(0, NQ)])
    iota = lax.iota(jnp.int32, 16)
    zeros16 = jnp.zeros((16,), jnp.int32)
    infv = jnp.full((16,), BIGF, jnp.float32)

    def qloop(j, carry):
        q = wid * QPW + j
        pltpu.sync_copy(d2_hbm.at[q], rowbuf)
        sv = selbuf[pl.ds(q, 16)]
        selfv = zeros16 + sv[0]

        def cbody(v, cnt):
            for u in range(2):
                off = (2 * v + u) * 16
                dv = rowbuf[pl.ds(off, 16)]
                mask = dv <= T_RAD
                plsc.store_compressed(candd.at[pl.ds(cnt, 16)], dv, mask=mask)
                iv = iota + off
                plsc.store_compressed(candi.at[pl.ds(cnt, 16)], iv, mask=mask)
                pc = plsc.all_reduce_population_count(mask)
                cnt = cnt + pc[0]
            return cnt

        cnt = lax.fori_loop(0, N // 32, cbody, jnp.zeros((), jnp.int32))
        cntv = zeros16 + cnt

        def load_vreg(v):
            cd = candd[pl.ds(v * 16, 16)]
            ci = candi[pl.ds(v * 16, 16)]
            lanei = iota + v * 16
            valid = lanei < cntv
            cd = jnp.where(valid, cd, infv)
            ci = jnp.where(valid, ci, selfv)
            return cd, ci

        def sel_fn():
            c0d, c0i = load_vreg(0)
            c1d, c1i = load_vreg(1)
            s0d, s0i = plsc.sort_key_val(c0d, c0i)
            s1d, s1i = plsc.sort_key_val(c1d, c1i)
            b0d, b0i, b1d, b1i = _merge16(s0d, s0i, s1d, s1i)

            def mbody(v, st):
                b0d, b0i, b1d, b1i = st
                cd, ci = load_vreg(v)
                sd, si = plsc.sort_key_val(cd, ci)
                lo1d, lo1i, hi1d, hi1i = _merge16(b1d, b1i, sd, si)
                b0d, b0i, hi2d, hi2i = _merge16(b0d, b0i, lo1d, lo1i)
                b1d, b1i, _, _ = _merge16(hi2d, hi2i, hi1d, hi1i)
                return (b0d, b0i, b1d, b1i)

            nv = (cnt + 15) // 16
            b0d, b0i, b1d, b1i = lax.fori_loop(2, nv, mbody, (b0d, b0i, b1d, b1i))
            return b0i, b1i

        def nosel_fn():
            _, c0i = load_vreg(0)
            _, c1i = load_vreg(1)
            return c0i, c1i

        b0i, b1i = lax.cond(cnt > N_REGION, sel_fn, nosel_fn)
        boff = (q // N_SAMPLED) * N
        idxbuf[0:16] = b0i + boff
        idxbuf[16:32] = b1i + boff
        pltpu.async_copy(a_hbm.at[idxbuf], hbuf, sem).wait()
        pltpu.sync_copy(hbuf, out_hbm.at[q])
        return carry

    lax.fori_loop(0, QPW, qloop, 0)


def _sel_gather(d2_flat, selidx_flat, a_flat):
    mesh = plsc.VectorSubcoreMesh(core_axis_name="c", subcore_axis_name="s")
    f = functools.partial(
        pl.kernel,
        mesh=mesh,
        compiler_params=pltpu.CompilerParams(needs_layout_passes=False),
        out_type=jax.ShapeDtypeStruct((NQ, N_REGION, 128), jnp.float32),
        scratch_types=[
            pltpu.VMEM((N,), jnp.float32),
            pltpu.VMEM((N + 32,), jnp.float32),
            pltpu.VMEM((N + 32,), jnp.int32),
            pltpu.VMEM((NQ + 16,), jnp.int32),
            pltpu.VMEM((N_REGION,), jnp.int32),
            pltpu.VMEM((N_REGION, 128), jnp.float32),
            pltpu.SemaphoreType.DMA,
        ],
    )(_sc_body)
    return f(d2_flat, selidx_flat, a_flat)


_QBLK = 64


def _mlp_kernel(h_ref, sxz_ref, w1x_ref, b1_ref, w2_ref, b2_ref, w3_ref, b3_ref, out_ref):
    dq = b1_ref[...] - sxz_ref[...] @ w1x_ref[...]
    x = h_ref[...][:, :, 0:64] + dq[:, None, :]
    x = jnp.where(x >= 0, x, 0.2 * x)
    x = x.reshape(_QBLK * N_REGION, 64)
    x = x @ w2_ref[...] + b2_ref[...]
    x = jnp.where(x >= 0, x, 0.2 * x)
    x = x @ w3_ref[...] + b3_ref[...]
    x = jnp.where(x >= 0, x, 0.2 * x)
    x = x.reshape(_QBLK, N_REGION, 128)
    out_ref[...] = jnp.max(x, axis=1)


def _mlp(H, sel_flat, W1x, b1, W2, b2, W3, b3):
    return pl.pallas_call(
        _mlp_kernel,
        grid=(NQ // _QBLK,),
        in_specs=[
            pl.BlockSpec((_QBLK, N_REGION, 128), lambda i: (i, 0, 0)),
            pl.BlockSpec((_QBLK, 3), lambda i: (i, 0)),
            pl.BlockSpec((3, 64), lambda i: (0, 0)),
            pl.BlockSpec((64,), lambda i: (0,)),
            pl.BlockSpec((64, 64), lambda i: (0, 0)),
            pl.BlockSpec((64,), lambda i: (0,)),
            pl.BlockSpec((64, 128), lambda i: (0, 0)),
            pl.BlockSpec((128,), lambda i: (0,)),
        ],
        out_specs=pl.BlockSpec((_QBLK, 128), lambda i: (i, 0)),
        out_shape=jax.ShapeDtypeStruct((NQ, 128), jnp.float32),
    )(H, sel_flat, W1x, b1, W2, b2, W3, b3)


def kernel(xyz, features, W1, b1, W2, b2, W3, b3):
    xyzT = jnp.transpose(xyz, (0, 2, 1))
    sel_idx, sel_xyzT = _fps(xyzT.reshape(B, 3, _NROW, 128), xyz)
    sel_xyz = jnp.transpose(sel_xyzT, (0, 2, 1))
    pf = jnp.concatenate([xyz, features], axis=-1).reshape(B * N, 67)
    W1p = jnp.pad(W1, ((0, 0), (0, 64)))
    A = _a_mm(pf, W1p)
    d2 = _d2mat(sel_xyz, xyzT)
    H = _sel_gather(d2.reshape(NQ, N), sel_idx.reshape(NQ), A)
    sel_feat = _mlp(H, sel_xyz.reshape(NQ, 3), W1[:3], b1, W2, b2, W3, b3)
    return (sel_xyz, sel_feat.reshape(B, N_SAMPLED, 128))


# speedup vs baseline: 12.0651x; 1.0561x over previous
import functools

import jax
import jax.numpy as jnp
from jax import lax
from jax.experimental import pallas as pl
from jax.experimental.pallas import tpu as pltpu
from jax.experimental.pallas import tpu_sc as plsc

N_SAMPLED = 1024
N_REGION = 32
B = 2
N = 8192
_NROW = 64
_SROW = 8

T_RAD = 0.04000000283122063
BIGF = 3.0e38

NQ = B * N_SAMPLED
_HALF = N // 2
_BOFF = N + 16
_CANDSZ = _BOFF + _HALF + 16
NWORK = 32
QPW = NQ // NWORK


def _fps_kernel(xyzT_ref, xyzr_ref, selidx_ref, selxyz_ref):
    iota_n = (
        lax.broadcasted_iota(jnp.int32, (_NROW, 128), 0) * 128
        + lax.broadcasted_iota(jnp.int32, (_NROW, 128), 1)
    )
    iota_s = (
        lax.broadcasted_iota(jnp.int32, (_SROW, 128), 0) * 128
        + lax.broadcasted_iota(jnp.int32, (_SROW, 128), 1)
    )
    x = [[xyzT_ref[b, c] for c in range(3)] for b in range(B)]

    def body(i, state):
        min_d2, far, sel, sxz = state
        new_md, new_far, new_sel, new_sxz = [], [], [], []
        for b in range(B):
            far_b = far[b]
            sel_b = jnp.where(iota_s == i, far_b, sel[b])
            cvec = xyzr_ref[b, pl.ds(far_b, 1), :]
            d2 = None
            sxz_b = []
            for c in range(3):
                cc = cvec[0:1, c : c + 1]
                sxz_b.append(jnp.where(iota_s == i, cc, sxz[b][c]))
                dc = x[b][c] - cc
                sq = dc * dc
                d2 = sq if d2 is None else d2 + sq
            md = jnp.minimum(min_d2[b], d2)
            mv, iv = md, iota_n
            k = _NROW // 2
            while k >= 1:
                av, ai = mv[0:k], iv[0:k]
                bv, bi = mv[k : 2 * k], iv[k : 2 * k]
                take = bv > av
                mv = jnp.where(take, bv, av)
                iv = jnp.where(take, bi, ai)
                k //= 2
            m = jnp.max(mv)
            cand = jnp.where(mv == m, iv, N)
            new_far.append(jnp.min(cand))
            new_md.append(md)
            new_sel.append(sel_b)
            new_sxz.append(sxz_b)
        return (new_md, new_far, new_sel, new_sxz)

    min_d2 = [jnp.full((_NROW, 128), 1e10, dtype=jnp.float32) for _ in range(B)]
    far = [jnp.zeros((), dtype=jnp.int32) for _ in range(B)]
    sel = [jnp.zeros((_SROW, 128), dtype=jnp.int32) for _ in range(B)]
    sxz = [[jnp.zeros((_SROW, 128), dtype=jnp.float32) for _ in range(3)] for _ in range(B)]
    _, _, sel, sxz = lax.fori_loop(0, N_SAMPLED, body, (min_d2, far, sel, sxz))
    for b in range(B):
        selidx_ref[b] = sel[b]
        for c in range(3):
            selxyz_ref[b, c] = sxz[b][c]


def _fps(xyzT4, xyz):
    sel, sxz = pl.pallas_call(
        _fps_kernel,
        out_shape=(
            jax.ShapeDtypeStruct((B, _SROW, 128), jnp.int32),
            jax.ShapeDtypeStruct((B, 3, _SROW, 128), jnp.float32),
        ),
    )(xyzT4, xyz)
    return sel.reshape(B, N_SAMPLED), sxz.reshape(B, 3, N_SAMPLED)


def _a_kernel(pf_ref, w1_ref, a_ref):
    a_ref[...] = pf_ref[...] @ w1_ref[...]


def _a_mm(pf, W1p):
    rows = pf.shape[0]
    blk = 2048
    return pl.pallas_call(
        _a_kernel,
        grid=(rows // blk,),
        in_specs=[
            pl.BlockSpec((blk, 67), lambda i: (i, 0)),
            pl.BlockSpec((67, 128), lambda i: (0, 0)),
        ],
        out_specs=pl.BlockSpec((blk, 128), lambda i: (i, 0)),
        out_shape=jax.ShapeDtypeStruct((rows, 128), jnp.float32),
    )(pf, W1p)


_BQ = 256
_BP = 1024


def _d2_kernel(sel_ref, xyzT_ref, out_ref):
    q = sel_ref[0]
    p = xyzT_ref[0]
    d2 = None
    for c in range(3):
        dc = q[:, c : c + 1] - p[c : c + 1, :]
        sq = dc * dc
        d2 = sq if d2 is None else d2 + sq
    out_ref[0] = d2


def _d2mat(sel_xyz, xyzT):
    return pl.pallas_call(
        _d2_kernel,
        grid=(B, N_SAMPLED // _BQ, N // _BP),
        in_specs=[
            pl.BlockSpec((1, _BQ, 3), lambda b, i, j: (b, i, 0)),
            pl.BlockSpec((1, 3, _BP), lambda b, i, j: (b, 0, j)),
        ],
        out_specs=pl.BlockSpec((1, _BQ, _BP), lambda b, i, j: (b, i, j)),
        out_shape=jax.ShapeDtypeStruct((B, N_SAMPLED, N), jnp.float32),
    )(sel_xyz, xyzT)


def _merge16(ad, ai, bd, bi):
    rd = lax.rev(bd, (0,))
    ri = lax.rev(bi, (0,))
    m = ad <= rd
    lod = jnp.where(m, ad, rd)
    loi = jnp.where(m, ai, ri)
    hid = jnp.where(m, rd, ad)
    hii = jnp.where(m, ri, ai)
    lod, loi = plsc.sort_key_val(lod, loi)
    hid, hii = plsc.sort_key_val(hid, hii)
    return lod, loi, hid, hii


def _sc_body(d2_hbm, selidx_hbm, a_hbm, out_hbm, rowbuf, candd, candi, selbuf, idxbuf, hbuf, sem):
    cid = lax.axis_index("c")
    sid = lax.axis_index("s")
    wid = sid * 2 + cid
    pltpu.sync_copy(selidx_hbm, selbuf.at[pl.ds(0, NQ)])
    iota = lax.iota(jnp.int32, 16)
    zeros16 = jnp.zeros((16,), jnp.int32)
    infv = jnp.full((16,), BIGF, jnp.float32)

    def qloop(j, carry):
        q = wid * QPW + j
        pltpu.sync_copy(d2_hbm.at[q], rowbuf)
        sv = selbuf[pl.ds(q, 16)]
        selfv = zeros16 + sv[0]

        def cbody(v, carry):
            ca, cb = carry
            for u in range(2):
                off = (2 * v + u) * 16
                dv = rowbuf[pl.ds(off, 16)]
                mask = dv <= T_RAD
                plsc.store_compressed(candd.at[pl.ds(ca, 16)], dv, mask=mask)
                plsc.store_compressed(candi.at[pl.ds(ca, 16)], iota + off, mask=mask)
                ca = ca + plsc.all_reduce_population_count(mask)[0]
                offb = _HALF + off
                dv2 = rowbuf[pl.ds(offb, 16)]
                mask2 = dv2 <= T_RAD
                plsc.store_compressed(candd.at[pl.ds(_BOFF + cb, 16)], dv2, mask=mask2)
                plsc.store_compressed(candi.at[pl.ds(_BOFF + cb, 16)], iota + offb, mask=mask2)
                cb = cb + plsc.all_reduce_population_count(mask2)[0]
            return (ca, cb)

        zero = jnp.zeros((), jnp.int32)
        ca, cb = lax.fori_loop(0, N // 64, cbody, (zero, zero))

        def sbody(k, carry):
            candd[pl.ds(ca + k * 16, 16)] = candd[pl.ds(_BOFF + k * 16, 16)]
            candi[pl.ds(ca + k * 16, 16)] = candi[pl.ds(_BOFF + k * 16, 16)]
            return carry

        lax.fori_loop(0, (cb + 15) // 16, sbody, 0)
        cnt = ca + cb
        cntv = zeros16 + cnt

        def load_vreg(v):
            cd = candd[pl.ds(v * 16, 16)]
            ci = candi[pl.ds(v * 16, 16)]
            lanei = iota + v * 16
            valid = lanei < cntv
            cd = jnp.where(valid, cd, infv)
            ci = jnp.where(valid, ci, selfv)
            return cd, ci

        def sel_fn():
            c0d, c0i = load_vreg(0)
            c1d, c1i = load_vreg(1)
            s0d, s0i = plsc.sort_key_val(c0d, c0i)
            s1d, s1i = plsc.sort_key_val(c1d, c1i)
            b0d, b0i, b1d, b1i = _merge16(s0d, s0i, s1d, s1i)

            def mbody(v, st):
                b0d, b0i, b1d, b1i = st
                cd, ci = load_vreg(v)
                sd, si = plsc.sort_key_val(cd, ci)
                lo1d, lo1i, hi1d, hi1i = _merge16(b1d, b1i, sd, si)
                b0d, b0i, hi2d, hi2i = _merge16(b0d, b0i, lo1d, lo1i)
                b1d, b1i, _, _ = _merge16(hi2d, hi2i, hi1d, hi1i)
                return (b0d, b0i, b1d, b1i)

            nv = (cnt + 15) // 16
            b0d, b0i, b1d, b1i = lax.fori_loop(2, nv, mbody, (b0d, b0i, b1d, b1i))
            return b0i, b1i

        def nosel_fn():
            _, c0i = load_vreg(0)
            _, c1i = load_vreg(1)
            return c0i, c1i

        b0i, b1i = lax.cond(cnt > N_REGION, sel_fn, nosel_fn)
        boff = (q // N_SAMPLED) * N
        idxbuf[0:16] = b0i + boff
        idxbuf[16:32] = b1i + boff
        pltpu.async_copy(a_hbm.at[idxbuf], hbuf, sem).wait()
        pltpu.sync_copy(hbuf, out_hbm.at[q])
        return carry

    lax.fori_loop(0, QPW, qloop, 0)


def _sel_gather(d2_flat, selidx_flat, a_flat):
    mesh = plsc.VectorSubcoreMesh(core_axis_name="c", subcore_axis_name="s")
    f = functools.partial(
        pl.kernel,
        mesh=mesh,
        compiler_params=pltpu.CompilerParams(needs_layout_passes=False),
        out_type=jax.ShapeDtypeStruct((NQ, N_REGION, 128), jnp.float32),
        scratch_types=[
            pltpu.VMEM((N,), jnp.float32),
            pltpu.VMEM((_CANDSZ,), jnp.float32),
            pltpu.VMEM((_CANDSZ,), jnp.int32),
            pltpu.VMEM((NQ + 16,), jnp.int32),
            pltpu.VMEM((N_REGION,), jnp.int32),
            pltpu.VMEM((N_REGION, 128), jnp.float32),
            pltpu.SemaphoreType.DMA,
        ],
    )(_sc_body)
    return f(d2_flat, selidx_flat, a_flat)


_QBLK = 64


def _mlp_kernel(h_ref, sxz_ref, w1x_ref, b1_ref, w2_ref, b2_ref, w3_ref, b3_ref, out_ref):
    dq = b1_ref[...] - sxz_ref[...] @ w1x_ref[...]
    x = h_ref[...][:, :, 0:64] + dq[:, None, :]
    x = jnp.where(x >= 0, x, 0.2 * x)
    x = x.reshape(_QBLK * N_REGION, 64)
    x = x @ w2_ref[...] + b2_ref[...]
    x = jnp.where(x >= 0, x, 0.2 * x)
    x = x @ w3_ref[...] + b3_ref[...]
    x = jnp.where(x >= 0, x, 0.2 * x)
    x = x.reshape(_QBLK, N_REGION, 128)
    out_ref[...] = jnp.max(x, axis=1)


def _mlp(H, sel_flat, W1x, b1, W2, b2, W3, b3):
    return pl.pallas_call(
        _mlp_kernel,
        grid=(NQ // _QBLK,),
        in_specs=[
            pl.BlockSpec((_QBLK, N_REGION, 128), lambda i: (i, 0, 0)),
            pl.BlockSpec((_QBLK, 3), lambda i: (i, 0)),
            pl.BlockSpec((3, 64), lambda i: (0, 0)),
            pl.BlockSpec((64,), lambda i: (0,)),
            pl.BlockSpec((64, 64), lambda i: (0, 0)),
            pl.BlockSpec((64,), lambda i: (0,)),
            pl.BlockSpec((64, 128), lambda i: (0, 0)),
            pl.BlockSpec((128,), lambda i: (0,)),
        ],
        out_specs=pl.BlockSpec((_QBLK, 128), lambda i: (i, 0)),
        out_shape=jax.ShapeDtypeStruct((NQ, 128), jnp.float32),
    )(H, sel_flat, W1x, b1, W2, b2, W3, b3)


def kernel(xyz, features, W1, b1, W2, b2, W3, b3):
    xyzT = jnp.transpose(xyz, (0, 2, 1))
    sel_idx, sel_xyzT = _fps(xyzT.reshape(B, 3, _NROW, 128), xyz)
    sel_xyz = jnp.transpose(sel_xyzT, (0, 2, 1))
    pf = jnp.concatenate([xyz, features], axis=-1).reshape(B * N, 67)
    W1p = jnp.pad(W1, ((0, 0), (0, 64)))
    A = _a_mm(pf, W1p)
    d2 = _d2mat(sel_xyz, xyzT)
    H = _sel_gather(d2.reshape(NQ, N), sel_idx.reshape(NQ), A)
    sel_feat = _mlp(H, sel_xyz.reshape(NQ, 3), W1[:3], b1, W2, b2, W3, b3)
    return (sel_xyz, sel_feat.reshape(B, N_SAMPLED, 128))
